# baseline (speedup 1.0000x reference)
import functools

import jax
import jax.numpy as jnp
from jax.experimental import pallas as pl
from jax.experimental.pallas import tpu as pltpu

_NCHUNK = 8


def _copy_dma(x_hbm, o_hbm, *sems):
    copies = [
        pltpu.make_async_copy(x_hbm.at[i], o_hbm.at[i], sems[i])
        for i in range(_NCHUNK)
    ]
    for c in copies:
        c.start()
    for c in copies:
        c.wait()


def kernel(x):
    m, n = x.shape
    assert m % _NCHUNK == 0
    xc = x.reshape(_NCHUNK, m // _NCHUNK, n)
    out = pl.pallas_call(
        _copy_dma,
        in_specs=[pl.BlockSpec(memory_space=pl.ANY)],
        out_specs=pl.BlockSpec(memory_space=pl.ANY),
        out_shape=jax.ShapeDtypeStruct(xc.shape, x.dtype),
        scratch_shapes=[pltpu.SemaphoreType.DMA] * _NCHUNK,
    )(xc)
    return out.reshape(m, n)


# speedup vs baseline: 51.7456x; 51.7456x over previous
import jax
import jax.numpy as jnp
from jax.experimental import pallas as pl
from jax.experimental.pallas import tpu as pltpu


def _copy_block(x_ref, o_ref):
    o_ref[...] = x_ref[...]


def kernel(x):
    m, n = x.shape
    bm = 5000 if m % 5000 == 0 else 8
    grid = (m // bm,)
    return pl.pallas_call(
        _copy_block,
        grid=grid,
        in_specs=[pl.BlockSpec((bm, n), lambda i: (i, 0))],
        out_specs=pl.BlockSpec((bm, n), lambda i: (i, 0)),
        out_shape=jax.ShapeDtypeStruct((m, n), x.dtype),
        compiler_params=pltpu.CompilerParams(
            dimension_semantics=("arbitrary",),
        ),
    )(x)
